# Initial kernel scaffold; baseline (speedup 1.0000x reference)
import functools

import jax
import jax.numpy as jnp
from jax import lax
from jax.experimental import pallas as pl
from jax.experimental.pallas import tpu as pltpu
from jax.experimental.pallas import tpu_sc as plsc

_N = 10000
_E = 320000
_D = 128
_NW = 32
_NCHUNK = 80
_CW = 125
_NPT = _N // 16

_mesh = plsc.VectorSubcoreMesh(core_axis_name="c", subcore_axis_name="s")


@functools.partial(
    pl.kernel,
    out_type=jax.ShapeDtypeStruct((2, _N, 16), jnp.float32),
    mesh=_mesh,
    scratch_types=[
        pltpu.VMEM((_NCHUNK, _CW), jnp.int32),
        pltpu.VMEM((_CW, 16), jnp.float32),
        pltpu.VMEM_SHARED((_N, 16), jnp.float32),
    ],
)
def _sc_degree(dst_hbm, ones_hbm, zeros_hbm, cnt_hbm, idx_v, ones_v, acc_sh):
    cid = lax.axis_index("c")
    sid = lax.axis_index("s")
    wid = cid * 16 + sid
    pltpu.sync_copy(dst_hbm.at[wid], idx_v)
    pltpu.sync_copy(ones_hbm, ones_v)
    pltpu.sync_copy(zeros_hbm, acc_sh.at[pl.ds(sid * _NPT, _NPT)])
    plsc.subcore_barrier()

    def body(j, carry):
        pltpu.sync_copy(ones_v, acc_sh.at[idx_v.at[j]], add=True)
        return carry

    lax.fori_loop(0, _NCHUNK, body, 0)
    plsc.subcore_barrier()
    pltpu.sync_copy(
        acc_sh.at[pl.ds(sid * _NPT, _NPT)],
        cnt_hbm.at[cid, pl.ds(sid * _NPT, _NPT)],
    )


@functools.partial(
    pl.kernel,
    out_type=jax.ShapeDtypeStruct((2, _N, _D), jnp.float32),
    mesh=_mesh,
    scratch_types=[
        pltpu.VMEM((_NCHUNK, _CW), jnp.int32),
        pltpu.VMEM((_NCHUNK, _CW), jnp.int32),
        pltpu.VMEM((_CW, _D), jnp.float32),
        pltpu.VMEM_SHARED((_N, _D), jnp.float32),
        pltpu.SemaphoreType.DMA,
    ],
)
def _sc_segsum(hs_hbm, src_hbm, dst_hbm, zeros_hbm, out_hbm,
               src_v, dst_v, rows_v, acc_sh, sem):
    cid = lax.axis_index("c")
    sid = lax.axis_index("s")
    wid = cid * 16 + sid
    pltpu.sync_copy(src_hbm.at[wid], src_v)
    pltpu.sync_copy(dst_hbm.at[wid], dst_v)
    pltpu.sync_copy(zeros_hbm, acc_sh.at[pl.ds(sid * _NPT, _NPT)])
    plsc.subcore_barrier()

    def body(j, carry):
        pltpu.async_copy(hs_hbm.at[src_v.at[j]], rows_v, sem).wait()
        pltpu.sync_copy(rows_v, acc_sh.at[dst_v.at[j]], add=True)
        return carry

    lax.fori_loop(0, _NCHUNK, body, 0)
    plsc.subcore_barrier()
    pltpu.sync_copy(
        acc_sh.at[pl.ds(sid * _NPT, _NPT)],
        out_hbm.at[cid, pl.ds(sid * _NPT, _NPT)],
    )


def _tc_first_body(cnt_ref, x_ref, w_ref, hs_ref, dinv_ref):
    c = cnt_ref[0, :, 0:1] + cnt_ref[1, :, 0:1]
    dinv = lax.rsqrt(1.0 + c)
    h = jnp.dot(x_ref[...], w_ref[...], preferred_element_type=jnp.float32)
    hs_ref[...] = h * dinv
    dinv_ref[...] = dinv


_tc_first = pl.pallas_call(
    _tc_first_body,
    out_shape=[
        jax.ShapeDtypeStruct((_N, _D), jnp.float32),
        jax.ShapeDtypeStruct((_N, 1), jnp.float32),
    ],
)


def _tc_mid_body(acc_ref, hs_ref, dinv_ref, b_ref, w_ref, out_ref):
    dinv = dinv_ref[...]
    pre = dinv * (acc_ref[0] + acc_ref[1] + hs_ref[...]) + b_ref[...]
    h = jnp.where(pre > 0, pre, jnp.expm1(pre))
    out_ref[...] = jnp.dot(
        h, w_ref[...], preferred_element_type=jnp.float32) * dinv


_tc_mid = pl.pallas_call(
    _tc_mid_body,
    out_shape=jax.ShapeDtypeStruct((_N, _D), jnp.float32),
)


def _tc_last_body(acc_ref, hs_ref, dinv_ref, b_ref, out_ref):
    pre = dinv_ref[...] * (acc_ref[0] + acc_ref[1] + hs_ref[...]) + b_ref[...]
    out_ref[...] = jnp.where(pre > 0, pre, jnp.expm1(pre))


_tc_last = pl.pallas_call(
    _tc_last_body,
    out_shape=jax.ShapeDtypeStruct((_N, _D), jnp.float32),
)


def kernel(x, edge_index, W1, b1, W2, b2):
    src = edge_index[0].reshape(_NW, _NCHUNK, _CW)
    dst = edge_index[1].reshape(_NW, _NCHUNK, _CW)
    ones16 = jnp.ones((_CW, 16), jnp.float32)
    zeros16 = jnp.zeros((_NPT, 16), jnp.float32)
    zerosD = jnp.zeros((_NPT, _D), jnp.float32)
    b1r = b1.reshape(1, _D)
    b2r = b2.reshape(1, _D)

    cnt = _sc_degree(dst, ones16, zeros16)
    hs1, dinv = _tc_first(cnt, x, W1)
    acc1 = _sc_segsum(hs1, src, dst, zerosD)
    hs2 = _tc_mid(acc1, hs1, dinv, b1r, W2)
    acc2 = _sc_segsum(hs2, src, dst, zerosD)
    return _tc_last(acc2, hs2, dinv, b2r)


# speedup vs baseline: 18.3895x; 18.3895x over previous
import functools

import jax
import jax.numpy as jnp
from jax import lax
from jax.experimental import pallas as pl
from jax.experimental.pallas import tpu as pltpu
from jax.experimental.pallas import tpu_sc as plsc

_N = 10000
_E = 320000
_D = 128
_NW = 32
_NCHUNK = 80
_CW = 125
_NP = 10240
_NPT = _NP // 16

_mesh = plsc.VectorSubcoreMesh(core_axis_name="c", subcore_axis_name="s")


@functools.partial(
    pl.kernel,
    out_type=jax.ShapeDtypeStruct((2, _NP, 16), jnp.float32),
    mesh=_mesh,
    scratch_types=[
        pltpu.VMEM((_NCHUNK, _CW), jnp.int32),
        pltpu.VMEM((_CW, 16), jnp.float32),
        pltpu.VMEM_SHARED((_NP, 16), jnp.float32),
    ],
)
def _sc_degree(dst_hbm, ones_hbm, zeros_hbm, cnt_hbm, idx_v, ones_v, acc_sh):
    cid = lax.axis_index("c")
    sid = lax.axis_index("s")
    wid = cid * 16 + sid
    pltpu.sync_copy(dst_hbm.at[wid], idx_v)
    pltpu.sync_copy(ones_hbm, ones_v)
    pltpu.sync_copy(zeros_hbm, acc_sh.at[pl.ds(sid * _NPT, _NPT)])
    plsc.subcore_barrier()

    def body(j, carry):
        pltpu.sync_copy(ones_v, acc_sh.at[idx_v.at[j]], add=True)
        return carry

    lax.fori_loop(0, _NCHUNK, body, 0)
    plsc.subcore_barrier()
    pltpu.sync_copy(
        acc_sh.at[pl.ds(sid * _NPT, _NPT)],
        cnt_hbm.at[cid, pl.ds(sid * _NPT, _NPT)],
    )


@functools.partial(
    pl.kernel,
    out_type=jax.ShapeDtypeStruct((2, _NP, _D), jnp.float32),
    mesh=_mesh,
    scratch_types=[
        pltpu.VMEM((_NCHUNK, _CW), jnp.int32),
        pltpu.VMEM((_NCHUNK, _CW), jnp.int32),
        pltpu.VMEM((_CW, _D), jnp.float32),
        pltpu.VMEM_SHARED((_NP, _D), jnp.float32),
        pltpu.SemaphoreType.DMA,
    ],
)
def _sc_segsum(hs_hbm, src_hbm, dst_hbm, zeros_hbm, out_hbm,
               src_v, dst_v, rows_v, acc_sh, sem):
    cid = lax.axis_index("c")
    sid = lax.axis_index("s")
    wid = cid * 16 + sid
    pltpu.sync_copy(src_hbm.at[wid], src_v)
    pltpu.sync_copy(dst_hbm.at[wid], dst_v)
    pltpu.sync_copy(zeros_hbm, acc_sh.at[pl.ds(sid * _NPT, _NPT)])
    plsc.subcore_barrier()

    def body(j, carry):
        pltpu.async_copy(hs_hbm.at[src_v.at[j]], rows_v, sem).wait()
        pltpu.sync_copy(rows_v, acc_sh.at[dst_v.at[j]], add=True)
        return carry

    lax.fori_loop(0, _NCHUNK, body, 0)
    plsc.subcore_barrier()
    pltpu.sync_copy(
        acc_sh.at[pl.ds(sid * _NPT, _NPT)],
        out_hbm.at[cid, pl.ds(sid * _NPT, _NPT)],
    )


def _tc_first_body(cnt_ref, x_ref, w_ref, hs_ref, dinv_ref):
    c = cnt_ref[0, :_N, 0:1] + cnt_ref[1, :_N, 0:1]
    dinv = lax.rsqrt(1.0 + c)
    h = jnp.dot(x_ref[...], w_ref[...], preferred_element_type=jnp.float32)
    hs_ref[...] = h * dinv
    dinv_ref[...] = dinv


_tc_first = pl.pallas_call(
    _tc_first_body,
    out_shape=[
        jax.ShapeDtypeStruct((_N, _D), jnp.float32),
        jax.ShapeDtypeStruct((_N, 1), jnp.float32),
    ],
)


def _tc_mid_body(acc_ref, hs_ref, dinv_ref, b_ref, w_ref, out_ref):
    dinv = dinv_ref[...]
    pre = dinv * (acc_ref[0, :_N] + acc_ref[1, :_N] + hs_ref[...]) + b_ref[...]
    h = jnp.where(pre > 0, pre, jnp.exp(pre) - 1.0)
    out_ref[...] = jnp.dot(
        h, w_ref[...], preferred_element_type=jnp.float32) * dinv


_tc_mid = pl.pallas_call(
    _tc_mid_body,
    out_shape=jax.ShapeDtypeStruct((_N, _D), jnp.float32),
)


def _tc_last_body(acc_ref, hs_ref, dinv_ref, b_ref, out_ref):
    pre = dinv_ref[...] * (acc_ref[0, :_N] + acc_ref[1, :_N] + hs_ref[...]) + b_ref[...]
    out_ref[...] = jnp.where(pre > 0, pre, jnp.exp(pre) - 1.0)


_tc_last = pl.pallas_call(
    _tc_last_body,
    out_shape=jax.ShapeDtypeStruct((_N, _D), jnp.float32),
)


def kernel(x, edge_index, W1, b1, W2, b2):
    src = edge_index[0].reshape(_NW, _NCHUNK, _CW)
    dst = edge_index[1].reshape(_NW, _NCHUNK, _CW)
    ones16 = jnp.ones((_CW, 16), jnp.float32)
    zeros16 = jnp.zeros((_NPT, 16), jnp.float32)
    zerosD = jnp.zeros((_NPT, _D), jnp.float32)
    b1r = b1.reshape(1, _D)
    b2r = b2.reshape(1, _D)

    cnt = _sc_degree(dst, ones16, zeros16)
    hs1, dinv = _tc_first(cnt, x, W1)
    acc1 = _sc_segsum(hs1, src, dst, zerosD)
    hs2 = _tc_mid(acc1, hs1, dinv, b1r, W2)
    acc2 = _sc_segsum(hs2, src, dst, zerosD)
    return _tc_last(acc2, hs2, dinv, b2r)


# speedup vs baseline: 25.9405x; 1.4106x over previous
import functools

import jax
import jax.numpy as jnp
from jax import lax
from jax.experimental import pallas as pl
from jax.experimental.pallas import tpu as pltpu
from jax.experimental.pallas import tpu_sc as plsc

_N = 10000
_E = 320000
_D = 128
_NW = 32
_NCHUNK = 80
_CW = 125
_GC = 40
_NP = 10240
_NPT = _NP // 16

_mesh = plsc.VectorSubcoreMesh(core_axis_name="c", subcore_axis_name="s")


@functools.partial(
    pl.kernel,
    out_type=jax.ShapeDtypeStruct((2, _NP, 16), jnp.float32),
    mesh=_mesh,
    scratch_types=[
        pltpu.VMEM((_NCHUNK, _CW), jnp.int32),
        pltpu.VMEM((_CW, 16), jnp.float32),
        pltpu.VMEM_SHARED((_NP, 16), jnp.float32),
    ],
)
def _sc_degree(dst_hbm, ones_hbm, zeros_hbm, cnt_hbm, idx_v, ones_v, acc_sh):
    cid = lax.axis_index("c")
    sid = lax.axis_index("s")
    wid = cid * 16 + sid
    pltpu.sync_copy(dst_hbm.at[wid], idx_v)
    pltpu.sync_copy(ones_hbm, ones_v)
    pltpu.sync_copy(zeros_hbm, acc_sh.at[pl.ds(sid * _NPT, _NPT)])
    plsc.subcore_barrier()

    def body(j, carry):
        pltpu.sync_copy(ones_v, acc_sh.at[idx_v.at[j]], add=True)
        return carry

    lax.fori_loop(0, _NCHUNK, body, 0)
    plsc.subcore_barrier()
    pltpu.sync_copy(
        acc_sh.at[pl.ds(sid * _NPT, _NPT)],
        cnt_hbm.at[cid, pl.ds(sid * _NPT, _NPT)],
    )


@functools.partial(
    pl.kernel,
    out_type=jax.ShapeDtypeStruct((2, _NP, _D), jnp.float32),
    mesh=_mesh,
    scratch_types=[
        pltpu.VMEM((_GC, _CW), jnp.int32),
        pltpu.VMEM((_GC, _CW), jnp.int32),
        pltpu.VMEM((_CW, _D), jnp.float32),
        pltpu.VMEM((_CW, _D), jnp.float32),
        pltpu.VMEM_SHARED((_NP, _D), jnp.float32),
        pltpu.SemaphoreType.DMA,
        pltpu.SemaphoreType.DMA,
    ],
)
def _sc_segsum(hs_hbm, src_hbm, dst_hbm, zeros_hbm, out_hbm,
               src_v, dst_v, rows0, rows1, acc_sh, sem0, sem1):
    cid = lax.axis_index("c")
    sid = lax.axis_index("s")
    wid = cid * 16 + sid
    pltpu.sync_copy(zeros_hbm, acc_sh.at[pl.ds(sid * _NPT, _NPT)])
    plsc.subcore_barrier()

    for g in range(_NCHUNK // _GC):
        pltpu.sync_copy(src_hbm.at[wid, pl.ds(g * _GC, _GC)], src_v)
        pltpu.sync_copy(dst_hbm.at[wid, pl.ds(g * _GC, _GC)], dst_v)
        pltpu.async_copy(hs_hbm.at[src_v.at[0]], rows0, sem0)

        def body(jj, carry):
            j0 = 2 * jj
            pltpu.async_copy(hs_hbm.at[src_v.at[j0 + 1]], rows1, sem1)
            pltpu.make_async_copy(hs_hbm.at[src_v.at[j0]], rows0, sem0).wait()
            pltpu.sync_copy(rows0, acc_sh.at[dst_v.at[j0]], add=True)

            @pl.when(jj < _GC // 2 - 1)
            def _():
                pltpu.async_copy(hs_hbm.at[src_v.at[j0 + 2]], rows0, sem0)

            pltpu.make_async_copy(
                hs_hbm.at[src_v.at[j0 + 1]], rows1, sem1).wait()
            pltpu.sync_copy(rows1, acc_sh.at[dst_v.at[j0 + 1]], add=True)
            return carry

        lax.fori_loop(0, _GC // 2, body, 0)
    plsc.subcore_barrier()
    pltpu.sync_copy(
        acc_sh.at[pl.ds(sid * _NPT, _NPT)],
        out_hbm.at[cid, pl.ds(sid * _NPT, _NPT)],
    )


def _tc_first_body(cnt_ref, x_ref, w_ref, hs_ref, dinv_ref):
    c = cnt_ref[0, :_N, 0:1] + cnt_ref[1, :_N, 0:1]
    dinv = lax.rsqrt(1.0 + c)
    h = jnp.dot(x_ref[...], w_ref[...], preferred_element_type=jnp.float32)
    hs_ref[...] = h * dinv
    dinv_ref[...] = dinv


_tc_first = pl.pallas_call(
    _tc_first_body,
    out_shape=[
        jax.ShapeDtypeStruct((_N, _D), jnp.float32),
        jax.ShapeDtypeStruct((_N, 1), jnp.float32),
    ],
)


def _tc_mid_body(acc_ref, hs_ref, dinv_ref, b_ref, w_ref, out_ref):
    dinv = dinv_ref[...]
    pre = dinv * (acc_ref[0, :_N] + acc_ref[1, :_N] + hs_ref[...]) + b_ref[...]
    h = jnp.where(pre > 0, pre, jnp.exp(pre) - 1.0)
    out_ref[...] = jnp.dot(
        h, w_ref[...], preferred_element_type=jnp.float32) * dinv


_tc_mid = pl.pallas_call(
    _tc_mid_body,
    out_shape=jax.ShapeDtypeStruct((_N, _D), jnp.float32),
)


def _tc_last_body(acc_ref, hs_ref, dinv_ref, b_ref, out_ref):
    pre = dinv_ref[...] * (acc_ref[0, :_N] + acc_ref[1, :_N] + hs_ref[...]) + b_ref[...]
    out_ref[...] = jnp.where(pre > 0, pre, jnp.exp(pre) - 1.0)


_tc_last = pl.pallas_call(
    _tc_last_body,
    out_shape=jax.ShapeDtypeStruct((_N, _D), jnp.float32),
)


def kernel(x, edge_index, W1, b1, W2, b2):
    src = edge_index[0].reshape(_NW, _NCHUNK, _CW)
    dst = edge_index[1].reshape(_NW, _NCHUNK, _CW)
    ones16 = jnp.ones((_CW, 16), jnp.float32)
    zeros16 = jnp.zeros((_NPT, 16), jnp.float32)
    zerosD = jnp.zeros((_NPT, _D), jnp.float32)
    b1r = b1.reshape(1, _D)
    b2r = b2.reshape(1, _D)

    cnt = _sc_degree(dst, ones16, zeros16)
    hs1, dinv = _tc_first(cnt, x, W1)
    acc1 = _sc_segsum(hs1, src, dst, zerosD)
    hs2 = _tc_mid(acc1, hs1, dinv, b1r, W2)
    acc2 = _sc_segsum(hs2, src, dst, zerosD)
    return _tc_last(acc2, hs2, dinv, b2r)
